# baseline (speedup 1.0000x reference)
import functools

import jax
import jax.numpy as jnp
from jax import lax
from jax.experimental import pallas as pl
from jax.experimental.pallas import tpu as pltpu
from jax.experimental.pallas import tpu_sc as plsc

DIM = 768
HID = 1024
E = 8
TOPK = 2
T = 2048

NC = 2
NS = 16
TB = T // (NC * NS)

_NEG = -1e30


def _logits_body(gw_ref, x_ref, l_ref):
    l_ref[...] = jax.lax.dot_general(
        gw_ref[...], x_ref[...], (((1,), (1,)), ((), ())),
        preferred_element_type=jnp.float32)


def _sc_router_body(l_hbm, bias_hbm, c_hbm, l_v, c_v, bias_v):
    wid = lax.axis_index("s") * NC + lax.axis_index("c")
    base = wid * TB
    for e in range(E):
        pltpu.sync_copy(l_hbm.at[e, pl.ds(base, TB)], l_v.at[e])
    pltpu.sync_copy(bias_hbm, bias_v)
    be = [bias_v[pl.ds(e * 16, 16)] for e in range(E)]

    for g in range(TB // 16):
        rows = [l_v[e, pl.ds(g * 16, 16)] for e in range(E)]
        m = rows[0]
        for r in rows[1:]:
            m = jnp.maximum(m, r)
        exs = [jnp.exp(r - m) for r in rows]
        ssum = exs[0]
        for ex in exs[1:]:
            ssum = ssum + ex
        inv = 1.0 / ssum
        sc = [ex * inv for ex in exs]
        b = [sc[e] + be[e] for e in range(E)]
        m1 = b[0]
        for r in b[1:]:
            m1 = jnp.maximum(m1, r)
        e1 = jnp.full((16,), E, jnp.int32)
        for e in range(E - 1, -1, -1):
            e1 = jnp.where(b[e] >= m1, e, e1)
        b2 = [jnp.where(e1 == e, _NEG, b[e]) for e in range(E)]
        m2 = b2[0]
        for r in b2[1:]:
            m2 = jnp.maximum(m2, r)
        e2 = jnp.full((16,), E, jnp.int32)
        for e in range(E - 1, -1, -1):
            e2 = jnp.where(b2[e] >= m2, e, e2)
        for e in range(E):
            keep = (e1 == e) | (e2 == e)
            c_v[e, pl.ds(g * 16, 16)] = jnp.where(keep, sc[e], 0.0)

    for e in range(E):
        pltpu.sync_copy(c_v.at[e], c_hbm.at[e, pl.ds(base, TB)])


def _moe_body(c_ref, x_ref, w1_ref, w2_ref, w3_ref, o_ref):
    e = pl.program_id(0)
    eh = (lax.broadcasted_iota(jnp.int32, (E, 1), 0) == e).astype(jnp.float32)
    ce = jax.lax.dot_general(
        c_ref[...], eh, (((0,), (0,)), ((), ())),
        preferred_element_type=jnp.float32)
    xe = (x_ref[...] * ce).astype(jnp.bfloat16)
    h1 = jax.lax.dot_general(
        xe, w1_ref[0].astype(jnp.bfloat16), (((1,), (0,)), ((), ())),
        preferred_element_type=jnp.float32)
    h3 = jax.lax.dot_general(
        xe, w3_ref[0].astype(jnp.bfloat16), (((1,), (0,)), ((), ())),
        preferred_element_type=jnp.float32)
    hh = ((h1 * jax.lax.logistic(h1)) * h3).astype(jnp.bfloat16)
    y = jax.lax.dot_general(
        hh, w2_ref[0].astype(jnp.bfloat16), (((1,), (0,)), ((), ())),
        preferred_element_type=jnp.float32)

    @pl.when(e == 0)
    def _init():
        o_ref[...] = y

    @pl.when(e > 0)
    def _acc():
        o_ref[...] += y


def kernel(x, gate_w, w1, w2, w3, expert_bias):
    bs, slen, dim = x.shape
    xt = x.reshape(T, dim)

    logits = pl.pallas_call(
        _logits_body,
        out_shape=jax.ShapeDtypeStruct((E, T), jnp.float32),
    )(gate_w, xt)

    bias16 = jnp.broadcast_to(expert_bias[:, None], (E, 16)).reshape(E * 16)

    mesh = plsc.VectorSubcoreMesh(core_axis_name="c", subcore_axis_name="s")
    sc_router = functools.partial(
        pl.kernel,
        mesh=mesh,
        out_type=jax.ShapeDtypeStruct((E, T), jnp.float32),
        scratch_types=[
            pltpu.VMEM((E, TB), jnp.float32),
            pltpu.VMEM((E, TB), jnp.float32),
            pltpu.VMEM((E * 16,), jnp.float32),
        ],
    )(_sc_router_body)
    c = sc_router(logits, bias16)

    out = pl.pallas_call(
        _moe_body,
        grid=(E,),
        in_specs=[
            pl.BlockSpec((E, T), lambda e: (0, 0)),
            pl.BlockSpec((T, dim), lambda e: (0, 0)),
            pl.BlockSpec((1, dim, HID), lambda e: (e, 0, 0)),
            pl.BlockSpec((1, HID, dim), lambda e: (e, 0, 0)),
            pl.BlockSpec((1, dim, HID), lambda e: (e, 0, 0)),
        ],
        out_specs=pl.BlockSpec((T, dim), lambda e: (0, 0)),
        out_shape=jax.ShapeDtypeStruct((T, dim), jnp.float32),
        compiler_params=pltpu.CompilerParams(
            dimension_semantics=("arbitrary",),
        ),
    )(c, xt, w1, w2, w3)

    return out.reshape(bs, slen, dim)


# speedup vs baseline: 1.3271x; 1.3271x over previous
import jax
import jax.numpy as jnp
from jax import lax
from jax.experimental import pallas as pl
from jax.experimental.pallas import tpu as pltpu

DIM = 768
HID = 1024
E = 8
TOPK = 2

_NEG = -1e30


def _moe_body(x_ref, gw_ref, bias_ref, w1_ref, w2_ref, w3_ref, o_ref, c_ref):
    e = pl.program_id(0)

    @pl.when(e == 0)
    def _router():
        logits = jax.lax.dot_general(
            x_ref[...], gw_ref[...], (((1,), (1,)), ((), ())),
            preferred_element_type=jnp.float32)
        m = jnp.max(logits, axis=1, keepdims=True)
        ex = jnp.exp(logits - m)
        scores = ex / jnp.sum(ex, axis=1, keepdims=True)
        b = scores + bias_ref[...]
        iota = lax.broadcasted_iota(jnp.int32, b.shape, 1)
        m1 = jnp.max(b, axis=1, keepdims=True)
        e1 = jnp.min(jnp.where(b >= m1, iota, E), axis=1, keepdims=True)
        b2 = jnp.where(iota == e1, _NEG, b)
        m2 = jnp.max(b2, axis=1, keepdims=True)
        e2 = jnp.min(jnp.where(b2 >= m2, iota, E), axis=1, keepdims=True)
        keep = (iota == e1) | (iota == e2)
        c_ref[...] = jnp.where(keep, scores, 0.0)

    c = c_ref[...]
    sel = lax.broadcasted_iota(jnp.int32, c.shape, 1) == e
    ce = jnp.sum(jnp.where(sel, c, 0.0), axis=1, keepdims=True)
    xe = (x_ref[...] * ce).astype(jnp.bfloat16)
    h1 = jax.lax.dot_general(
        xe, w1_ref[0].astype(jnp.bfloat16), (((1,), (0,)), ((), ())),
        preferred_element_type=jnp.float32)
    h3 = jax.lax.dot_general(
        xe, w3_ref[0].astype(jnp.bfloat16), (((1,), (0,)), ((), ())),
        preferred_element_type=jnp.float32)
    hh = ((h1 * jax.lax.logistic(h1)) * h3).astype(jnp.bfloat16)
    y = jax.lax.dot_general(
        hh, w2_ref[0].astype(jnp.bfloat16), (((1,), (0,)), ((), ())),
        preferred_element_type=jnp.float32)

    @pl.when(e == 0)
    def _init():
        o_ref[...] = y

    @pl.when(e > 0)
    def _acc():
        o_ref[...] += y


def kernel(x, gate_w, w1, w2, w3, expert_bias):
    bs, slen, dim = x.shape
    T = bs * slen
    xt = x.reshape(T, dim)

    out = pl.pallas_call(
        _moe_body,
        grid=(E,),
        in_specs=[
            pl.BlockSpec((T, dim), lambda e: (0, 0)),
            pl.BlockSpec((E, dim), lambda e: (0, 0)),
            pl.BlockSpec((1, E), lambda e: (0, 0)),
            pl.BlockSpec((1, dim, HID), lambda e: (e, 0, 0)),
            pl.BlockSpec((1, HID, dim), lambda e: (e, 0, 0)),
            pl.BlockSpec((1, dim, HID), lambda e: (e, 0, 0)),
        ],
        out_specs=pl.BlockSpec((T, dim), lambda e: (0, 0)),
        out_shape=jax.ShapeDtypeStruct((T, dim), jnp.float32),
        scratch_shapes=[pltpu.VMEM((T, E), jnp.float32)],
        compiler_params=pltpu.CompilerParams(
            dimension_semantics=("arbitrary",),
        ),
    )(xt, gate_w, expert_bias.reshape(1, E), w1, w2, w3)

    return out.reshape(bs, slen, dim)
